# Initial kernel scaffold; baseline (speedup 1.0000x reference)
import jax
import jax.numpy as jnp
from jax.experimental import pallas as pl


def kernel(x, edge_index, W1, b1, W2, b2, W3, b3):
    raise NotImplementedError("write your pallas kernel here")


# speedup vs baseline: 3.1623x; 3.1623x over previous
import functools

import jax
import jax.numpy as jnp
from jax import lax
from jax.experimental import pallas as pl
from jax.experimental.pallas import tpu as pltpu
from jax.experimental.pallas import tpu_sc as plsc

N_NODES = 10000
N_EDGES = 160000
IN_CH = 256
HID_CH = 256
OUT_CH = 128
K_PROP = 10
ALPHA = 0.1

NC = 2
NS = 16
NT = NC * NS
NPAD = 10240
CH = 128
EPT = 5120
NCHUNK = EPT // CH
ROWS_PER_TILE = NPAD // NS
RB = 1280

_f32 = jnp.float32
_i32 = jnp.int32

_MESH = plsc.VectorSubcoreMesh(core_axis_name="c", subcore_axis_name="s")


def _fill_rows(ref, nrows, width, value):
  def body(i, carry):
    for j in range(width // 16):
      ref[i, 16 * j:16 * (j + 1)] = jnp.full((16,), value, _f32)
    return carry

  lax.fori_loop(0, nrows, body, 0)


@functools.partial(
    pl.kernel,
    out_type=jax.ShapeDtypeStruct((NC, 2, NPAD, OUT_CH), _f32),
    mesh=_MESH,
    scratch_types=[
        pltpu.VMEM_SHARED((NPAD, OUT_CH), _f32),
        pltpu.VMEM((CH, OUT_CH), _f32),
        pltpu.VMEM((NCHUNK, CH), _i32),
    ],
)
def _deg_kernel(srci_hbm, dsti_hbm, out_hbm, agg_sh, buf, idxv):
  c = lax.axis_index("c")
  s = lax.axis_index("s")
  tid = c * NS + s
  base = s * ROWS_PER_TILE

  for which, idx_hbm in ((0, srci_hbm), (1, dsti_hbm)):
    _fill_rows(buf, CH, OUT_CH, 0.0)
    for k in range(ROWS_PER_TILE // CH):
      pltpu.sync_copy(buf, agg_sh.at[pl.ds(base + k * CH, CH)])
    plsc.subcore_barrier()
    _fill_rows(buf, CH, OUT_CH, 1.0)
    pltpu.sync_copy(idx_hbm.at[tid], idxv)

    def body(j, carry):
      pltpu.sync_copy(buf, agg_sh.at[idxv.at[j]], add=True)
      return carry

    lax.fori_loop(0, NCHUNK, body, 0)
    plsc.subcore_barrier()
    for k in range(ROWS_PER_TILE // CH):
      sl = pl.ds(base + k * CH, CH)
      pltpu.sync_copy(agg_sh.at[sl], out_hbm.at[c].at[which].at[sl])
    plsc.subcore_barrier()


@functools.partial(
    pl.kernel,
    out_type=jax.ShapeDtypeStruct((NC, NPAD, OUT_CH), _f32),
    mesh=_MESH,
    scratch_types=[
        pltpu.VMEM_SHARED((NPAD, OUT_CH), _f32),
        pltpu.VMEM((CH, OUT_CH), _f32),
        pltpu.VMEM((NCHUNK, CH), _i32),
        pltpu.VMEM((NCHUNK, CH), _i32),
        pltpu.SemaphoreType.DMA,
    ],
)
def _step_kernel(g_hbm, srci_hbm, dsti_hbm, out_hbm, agg_sh, rows_v, srcv,
                 dstv, sem):
  c = lax.axis_index("c")
  s = lax.axis_index("s")
  tid = c * NS + s
  _fill_rows(rows_v, CH, OUT_CH, 0.0)
  base = s * ROWS_PER_TILE
  for k in range(ROWS_PER_TILE // CH):
    pltpu.sync_copy(rows_v, agg_sh.at[pl.ds(base + k * CH, CH)])
  plsc.subcore_barrier()
  pltpu.sync_copy(srci_hbm.at[tid], srcv)
  pltpu.sync_copy(dsti_hbm.at[tid], dstv)

  def body(j, carry):
    pltpu.async_copy(g_hbm.at[srcv.at[j]], rows_v, sem).wait()
    pltpu.sync_copy(rows_v, agg_sh.at[dstv.at[j]], add=True)
    return carry

  lax.fori_loop(0, NCHUNK, body, 0)
  plsc.subcore_barrier()
  for k in range(ROWS_PER_TILE // CH):
    sl = pl.ds(base + k * CH, CH)
    pltpu.sync_copy(agg_sh.at[sl], out_hbm.at[c].at[sl])


def _mlp_body(x_ref, w1_ref, b1_ref, w2_ref, b2_ref, w3_ref, b3_ref, o_ref):
  h = jnp.dot(x_ref[...], w1_ref[...], preferred_element_type=_f32)
  h = jnp.maximum(h + b1_ref[...], 0.0)
  h = jnp.dot(h, w2_ref[...], preferred_element_type=_f32)
  h = jnp.maximum(h + b2_ref[...], 0.0)
  o_ref[...] = jnp.dot(h, w3_ref[...], preferred_element_type=_f32) + b3_ref[...]


def _mlp(x_pad, W1, b1, W2, b2, W3, b3):
  nb = NPAD // RB
  full = lambda shape: pl.BlockSpec(shape, lambda i: tuple(0 for _ in shape))
  return pl.pallas_call(
      _mlp_body,
      grid=(nb,),
      in_specs=[
          pl.BlockSpec((RB, IN_CH), lambda i: (i, 0)),
          full((IN_CH, HID_CH)), full((1, HID_CH)),
          full((HID_CH, HID_CH)), full((1, HID_CH)),
          full((HID_CH, OUT_CH)), full((1, OUT_CH)),
      ],
      out_specs=pl.BlockSpec((RB, OUT_CH), lambda i: (i, 0)),
      out_shape=jax.ShapeDtypeStruct((NPAD, OUT_CH), _f32),
  )(x_pad, W1, b1, W2, b2, W3, b3)


def _prep_body(h_ref, deg_ref, g0_ref, c0_ref, c1_ref, fd_ref, ah0_ref):
  h = h_ref[...]
  dsrc = deg_ref[0, 0, :, 0:1] + deg_ref[1, 0, :, 0:1]
  ddst = deg_ref[0, 1, :, 0:1] + deg_ref[1, 1, :, 0:1]
  rs = lax.rsqrt(jnp.maximum(dsrc, 1.0))
  rd = lax.rsqrt(jnp.maximum(ddst, 1.0))
  g0_ref[...] = rs * h
  ah0_ref[...] = ALPHA * h
  c0_ref[...] = (ALPHA * rs) * h
  c1_ref[...] = jnp.broadcast_to((1.0 - ALPHA) * rs * rd, h.shape)
  fd_ref[...] = jnp.broadcast_to((1.0 - ALPHA) * rd, h.shape)


def _prep(h_pad, degs):
  nb = NPAD // RB
  sds = jax.ShapeDtypeStruct((NPAD, OUT_CH), _f32)
  return pl.pallas_call(
      _prep_body,
      grid=(nb,),
      in_specs=[
          pl.BlockSpec((RB, OUT_CH), lambda i: (i, 0)),
          pl.BlockSpec((NC, 2, RB, OUT_CH), lambda i: (0, 0, i, 0)),
      ],
      out_specs=[pl.BlockSpec((RB, OUT_CH), lambda i: (i, 0))] * 5,
      out_shape=[sds] * 5,
  )(h_pad, degs)


def _combine_body(s_ref, c1_ref, c0_ref, g_ref):
  g_ref[...] = c1_ref[...] * (s_ref[0] + s_ref[1]) + c0_ref[...]


def _combine(S2, c1f, c0):
  nb = NPAD // RB
  return pl.pallas_call(
      _combine_body,
      grid=(nb,),
      in_specs=[
          pl.BlockSpec((NC, RB, OUT_CH), lambda i: (0, i, 0)),
          pl.BlockSpec((RB, OUT_CH), lambda i: (i, 0)),
          pl.BlockSpec((RB, OUT_CH), lambda i: (i, 0)),
      ],
      out_specs=pl.BlockSpec((RB, OUT_CH), lambda i: (i, 0)),
      out_shape=jax.ShapeDtypeStruct((NPAD, OUT_CH), _f32),
  )(S2, c1f, c0)


def _final_body(s_ref, fd_ref, ah0_ref, o_ref):
  z = fd_ref[...] * (s_ref[0] + s_ref[1]) + ah0_ref[...]
  m = jnp.max(z, axis=1, keepdims=True)
  lse = jnp.log(jnp.sum(jnp.exp(z - m), axis=1, keepdims=True)) + m
  o_ref[...] = z - lse


def _final(S2, fd, ah0):
  fb = 1000
  nb = N_NODES // fb
  return pl.pallas_call(
      _final_body,
      grid=(nb,),
      in_specs=[
          pl.BlockSpec((NC, fb, OUT_CH), lambda i: (0, i, 0)),
          pl.BlockSpec((fb, OUT_CH), lambda i: (i, 0)),
          pl.BlockSpec((fb, OUT_CH), lambda i: (i, 0)),
      ],
      out_specs=pl.BlockSpec((fb, OUT_CH), lambda i: (i, 0)),
      out_shape=jax.ShapeDtypeStruct((N_NODES, OUT_CH), _f32),
  )(S2, fd, ah0)


def kernel(x, edge_index, W1, b1, W2, b2, W3, b3):
  x_pad = jnp.concatenate(
      [x, jnp.zeros((NPAD - N_NODES, IN_CH), _f32)], axis=0)
  pad_n = NT * EPT - N_EDGES
  src = edge_index[0].astype(_i32)
  dst = edge_index[1].astype(_i32)
  pad_idx = jnp.full((pad_n,), N_NODES, _i32)
  srci = jnp.concatenate([src, pad_idx]).reshape(NT, NCHUNK, CH)
  dsti = jnp.concatenate([dst, pad_idx]).reshape(NT, NCHUNK, CH)

  h_pad = _mlp(x_pad, W1, b1.reshape(1, HID_CH), W2, b2.reshape(1, HID_CH),
               W3, b3.reshape(1, OUT_CH))
  degs = _deg_kernel(srci, dsti)
  g, c0, c1f, fd, ah0 = _prep(h_pad, degs)
  for _ in range(K_PROP - 1):
    S2 = _step_kernel(g, srci, dsti)
    g = _combine(S2, c1f, c0)
  S2 = _step_kernel(g, srci, dsti)
  return _final(S2, fd, ah0)


# speedup vs baseline: 3.3923x; 1.0727x over previous
import functools

import jax
import jax.numpy as jnp
from jax import lax
from jax.experimental import pallas as pl
from jax.experimental.pallas import tpu as pltpu
from jax.experimental.pallas import tpu_sc as plsc

N_NODES = 10000
N_EDGES = 160000
IN_CH = 256
HID_CH = 256
OUT_CH = 128
K_PROP = 10
ALPHA = 0.1

NC = 2
NS = 16
NT = NC * NS
NPAD = 10240
CH = 128
EPT = 5120
NCHUNK = EPT // CH
ROWS_PER_TILE = NPAD // NS
RB = 1280

_f32 = jnp.float32
_i32 = jnp.int32

_MESH = plsc.VectorSubcoreMesh(core_axis_name="c", subcore_axis_name="s")


def _fill_rows(ref, nrows, width, value):
  def body(i, carry):
    for j in range(width // 16):
      ref[i, 16 * j:16 * (j + 1)] = jnp.full((16,), value, _f32)
    return carry

  lax.fori_loop(0, nrows, body, 0)


@functools.partial(
    pl.kernel,
    out_type=jax.ShapeDtypeStruct((NC, 2, NPAD, OUT_CH), _f32),
    mesh=_MESH,
    scratch_types=[
        pltpu.VMEM_SHARED((NPAD, OUT_CH), _f32),
        pltpu.VMEM((CH, OUT_CH), _f32),
        pltpu.VMEM((NCHUNK, CH), _i32),
    ],
)
def _deg_kernel(srci_hbm, dsti_hbm, zrs_hbm, out_hbm, agg_sh, buf, idxv):
  c = lax.axis_index("c")
  s = lax.axis_index("s")
  tid = c * NS + s
  base = s * ROWS_PER_TILE
  _fill_rows(buf, CH, OUT_CH, 1.0)

  for which, idx_hbm in ((0, srci_hbm), (1, dsti_hbm)):
    pltpu.sync_copy(zrs_hbm.at[pl.ds(base, ROWS_PER_TILE)],
                    agg_sh.at[pl.ds(base, ROWS_PER_TILE)])
    plsc.subcore_barrier()
    pltpu.sync_copy(idx_hbm.at[tid], idxv)

    def body(j, carry):
      pltpu.sync_copy(buf, agg_sh.at[idxv.at[j]], add=True)
      return carry

    lax.fori_loop(0, NCHUNK, body, 0)
    plsc.subcore_barrier()
    pltpu.sync_copy(agg_sh.at[pl.ds(base, ROWS_PER_TILE)],
                    out_hbm.at[c].at[which].at[pl.ds(base, ROWS_PER_TILE)])
    plsc.subcore_barrier()


@functools.partial(
    pl.kernel,
    out_type=jax.ShapeDtypeStruct((NC, NPAD, OUT_CH), _f32),
    mesh=_MESH,
    scratch_types=[
        pltpu.VMEM_SHARED((NPAD, OUT_CH), _f32),
        pltpu.VMEM((CH, OUT_CH), _f32),
        pltpu.VMEM((CH, OUT_CH), _f32),
        pltpu.VMEM((NCHUNK, CH), _i32),
        pltpu.VMEM((NCHUNK, CH), _i32),
        pltpu.SemaphoreType.DMA,
        pltpu.SemaphoreType.DMA,
    ],
)
def _step_kernel(g_hbm, srci_hbm, dsti_hbm, zrs_hbm, out_hbm, agg_sh, b0, b1,
                 srcv, dstv, sem0, sem1):
  c = lax.axis_index("c")
  s = lax.axis_index("s")
  tid = c * NS + s
  base = s * ROWS_PER_TILE
  pltpu.sync_copy(zrs_hbm.at[pl.ds(base, ROWS_PER_TILE)],
                  agg_sh.at[pl.ds(base, ROWS_PER_TILE)])
  pltpu.sync_copy(srci_hbm.at[tid], srcv)
  pltpu.sync_copy(dsti_hbm.at[tid], dstv)
  plsc.subcore_barrier()

  bufs = (b0, b1)
  sems = (sem0, sem1)
  for b in range(2):
    pltpu.async_copy(g_hbm.at[srcv.at[b]], bufs[b], sems[b])

  def body(grp, carry):
    for b in range(2):
      j = grp * 2 + b
      pltpu.make_async_copy(g_hbm.at[srcv.at[j]], bufs[b], sems[b]).wait()
      pltpu.sync_copy(bufs[b], agg_sh.at[dstv.at[j]], add=True)

      @pl.when(j + 2 < NCHUNK)
      def _():
        pltpu.async_copy(g_hbm.at[srcv.at[j + 2]], bufs[b], sems[b])
    return carry

  lax.fori_loop(0, NCHUNK // 2, body, 0)
  plsc.subcore_barrier()
  pltpu.sync_copy(agg_sh.at[pl.ds(base, ROWS_PER_TILE)],
                  out_hbm.at[c].at[pl.ds(base, ROWS_PER_TILE)])


def _mlp_body(x_ref, w1_ref, b1_ref, w2_ref, b2_ref, w3_ref, b3_ref, o_ref):
  h = jnp.dot(x_ref[...], w1_ref[...], preferred_element_type=_f32)
  h = jnp.maximum(h + b1_ref[...], 0.0)
  h = jnp.dot(h, w2_ref[...], preferred_element_type=_f32)
  h = jnp.maximum(h + b2_ref[...], 0.0)
  o_ref[...] = jnp.dot(h, w3_ref[...], preferred_element_type=_f32) + b3_ref[...]


def _mlp(x_pad, W1, b1, W2, b2, W3, b3):
  nb = NPAD // RB
  full = lambda shape: pl.BlockSpec(shape, lambda i: tuple(0 for _ in shape))
  return pl.pallas_call(
      _mlp_body,
      grid=(nb,),
      in_specs=[
          pl.BlockSpec((RB, IN_CH), lambda i: (i, 0)),
          full((IN_CH, HID_CH)), full((1, HID_CH)),
          full((HID_CH, HID_CH)), full((1, HID_CH)),
          full((HID_CH, OUT_CH)), full((1, OUT_CH)),
      ],
      out_specs=pl.BlockSpec((RB, OUT_CH), lambda i: (i, 0)),
      out_shape=jax.ShapeDtypeStruct((NPAD, OUT_CH), _f32),
  )(x_pad, W1, b1, W2, b2, W3, b3)


def _prep_body(h_ref, deg_ref, g0_ref, c0_ref, c1_ref, fd_ref, ah0_ref):
  h = h_ref[...]
  dsrc = deg_ref[0, 0, :, 0:1] + deg_ref[1, 0, :, 0:1]
  ddst = deg_ref[0, 1, :, 0:1] + deg_ref[1, 1, :, 0:1]
  rs = lax.rsqrt(jnp.maximum(dsrc, 1.0))
  rd = lax.rsqrt(jnp.maximum(ddst, 1.0))
  g0_ref[...] = rs * h
  ah0_ref[...] = ALPHA * h
  c0_ref[...] = (ALPHA * rs) * h
  c1_ref[...] = jnp.broadcast_to((1.0 - ALPHA) * rs * rd, h.shape)
  fd_ref[...] = jnp.broadcast_to((1.0 - ALPHA) * rd, h.shape)


def _prep(h_pad, degs):
  nb = NPAD // RB
  sds = jax.ShapeDtypeStruct((NPAD, OUT_CH), _f32)
  return pl.pallas_call(
      _prep_body,
      grid=(nb,),
      in_specs=[
          pl.BlockSpec((RB, OUT_CH), lambda i: (i, 0)),
          pl.BlockSpec((NC, 2, RB, OUT_CH), lambda i: (0, 0, i, 0)),
      ],
      out_specs=[pl.BlockSpec((RB, OUT_CH), lambda i: (i, 0))] * 5,
      out_shape=[sds] * 5,
  )(h_pad, degs)


def _combine_body(s_ref, c1_ref, c0_ref, g_ref):
  g_ref[...] = c1_ref[...] * (s_ref[0] + s_ref[1]) + c0_ref[...]


def _combine(S2, c1f, c0):
  nb = NPAD // RB
  return pl.pallas_call(
      _combine_body,
      grid=(nb,),
      in_specs=[
          pl.BlockSpec((NC, RB, OUT_CH), lambda i: (0, i, 0)),
          pl.BlockSpec((RB, OUT_CH), lambda i: (i, 0)),
          pl.BlockSpec((RB, OUT_CH), lambda i: (i, 0)),
      ],
      out_specs=pl.BlockSpec((RB, OUT_CH), lambda i: (i, 0)),
      out_shape=jax.ShapeDtypeStruct((NPAD, OUT_CH), _f32),
  )(S2, c1f, c0)


def _final_body(s_ref, fd_ref, ah0_ref, o_ref):
  z = fd_ref[...] * (s_ref[0] + s_ref[1]) + ah0_ref[...]
  m = jnp.max(z, axis=1, keepdims=True)
  lse = jnp.log(jnp.sum(jnp.exp(z - m), axis=1, keepdims=True)) + m
  o_ref[...] = z - lse


def _final(S2, fd, ah0):
  fb = 1000
  nb = N_NODES // fb
  return pl.pallas_call(
      _final_body,
      grid=(nb,),
      in_specs=[
          pl.BlockSpec((NC, fb, OUT_CH), lambda i: (0, i, 0)),
          pl.BlockSpec((fb, OUT_CH), lambda i: (i, 0)),
          pl.BlockSpec((fb, OUT_CH), lambda i: (i, 0)),
      ],
      out_specs=pl.BlockSpec((fb, OUT_CH), lambda i: (i, 0)),
      out_shape=jax.ShapeDtypeStruct((N_NODES, OUT_CH), _f32),
  )(S2, fd, ah0)


def kernel(x, edge_index, W1, b1, W2, b2, W3, b3):
  x_pad = jnp.concatenate(
      [x, jnp.zeros((NPAD - N_NODES, IN_CH), _f32)], axis=0)
  pad_n = NT * EPT - N_EDGES
  src = edge_index[0].astype(_i32)
  dst = edge_index[1].astype(_i32)
  pad_idx = jnp.full((pad_n,), N_NODES, _i32)
  srci = jnp.concatenate([src, pad_idx]).reshape(NT, NCHUNK, CH)
  dsti = jnp.concatenate([dst, pad_idx]).reshape(NT, NCHUNK, CH)
  zrs = jnp.zeros((NPAD, OUT_CH), _f32)

  h_pad = _mlp(x_pad, W1, b1.reshape(1, HID_CH), W2, b2.reshape(1, HID_CH),
               W3, b3.reshape(1, OUT_CH))
  degs = _deg_kernel(srci, dsti, zrs)
  g, c0, c1f, fd, ah0 = _prep(h_pad, degs)
  for _ in range(K_PROP - 1):
    S2 = _step_kernel(g, srci, dsti, zrs)
    g = _combine(S2, c1f, c0)
  S2 = _step_kernel(g, srci, dsti, zrs)
  return _final(S2, fd, ah0)


# speedup vs baseline: 3.3934x; 1.0003x over previous
import functools

import jax
import jax.numpy as jnp
from jax import lax
from jax.experimental import pallas as pl
from jax.experimental.pallas import tpu as pltpu
from jax.experimental.pallas import tpu_sc as plsc

N_NODES = 10000
N_EDGES = 160000
IN_CH = 256
HID_CH = 256
OUT_CH = 128
K_PROP = 10
ALPHA = 0.1

NC = 2
NS = 16
NT = NC * NS
NPAD = 10240
CH = 128
EPT = 5120
NCHUNK = EPT // CH
ROWS_PER_TILE = NPAD // NS
RB = 1280

_f32 = jnp.float32
_i32 = jnp.int32

_MESH = plsc.VectorSubcoreMesh(core_axis_name="c", subcore_axis_name="s")


def _fill_rows(ref, nrows, width, value):
  def body(i, carry):
    for j in range(width // 16):
      ref[i, 16 * j:16 * (j + 1)] = jnp.full((16,), value, _f32)
    return carry

  lax.fori_loop(0, nrows, body, 0)


@functools.partial(
    pl.kernel,
    out_type=jax.ShapeDtypeStruct((NC, 2, NPAD, OUT_CH), _f32),
    mesh=_MESH,
    scratch_types=[
        pltpu.VMEM_SHARED((NPAD, OUT_CH), _f32),
        pltpu.VMEM((CH, OUT_CH), _f32),
        pltpu.VMEM((NCHUNK, CH), _i32),
        pltpu.SemaphoreType.DMA,
    ],
)
def _deg_kernel(srci_hbm, dsti_hbm, zrs_hbm, out_hbm, agg_sh, buf, idxv, sem):
  c = lax.axis_index("c")
  s = lax.axis_index("s")
  tid = c * NS + s
  base = s * ROWS_PER_TILE
  _fill_rows(buf, CH, OUT_CH, 1.0)

  for which, idx_hbm in ((0, srci_hbm), (1, dsti_hbm)):
    pltpu.sync_copy(zrs_hbm.at[pl.ds(base, ROWS_PER_TILE)],
                    agg_sh.at[pl.ds(base, ROWS_PER_TILE)])
    plsc.subcore_barrier()
    pltpu.sync_copy(idx_hbm.at[tid], idxv)

    def body(j, carry):
      pltpu.async_copy(buf, agg_sh.at[idxv.at[j]], sem, add=True)
      return carry

    lax.fori_loop(0, NCHUNK, body, 0)

    def drain(j, carry):
      pltpu.make_async_copy(buf, agg_sh.at[idxv.at[j]], sem).wait()
      return carry

    lax.fori_loop(0, NCHUNK, drain, 0)
    plsc.subcore_barrier()
    pltpu.sync_copy(agg_sh.at[pl.ds(base, ROWS_PER_TILE)],
                    out_hbm.at[c].at[which].at[pl.ds(base, ROWS_PER_TILE)])
    plsc.subcore_barrier()


@functools.partial(
    pl.kernel,
    out_type=jax.ShapeDtypeStruct((NC, NPAD, OUT_CH), _f32),
    mesh=_MESH,
    scratch_types=[
        pltpu.VMEM_SHARED((NPAD, OUT_CH), _f32),
        pltpu.VMEM((CH, OUT_CH), _f32),
        pltpu.VMEM((CH, OUT_CH), _f32),
        pltpu.VMEM((NCHUNK, CH), _i32),
        pltpu.VMEM((NCHUNK, CH), _i32),
        pltpu.SemaphoreType.DMA,
        pltpu.SemaphoreType.DMA,
    ],
)
def _step_kernel(g_hbm, srci_hbm, dsti_hbm, zrs_hbm, out_hbm, agg_sh, b0, b1,
                 srcv, dstv, sem0, sem1):
  c = lax.axis_index("c")
  s = lax.axis_index("s")
  tid = c * NS + s
  base = s * ROWS_PER_TILE
  pltpu.sync_copy(zrs_hbm.at[pl.ds(base, ROWS_PER_TILE)],
                  agg_sh.at[pl.ds(base, ROWS_PER_TILE)])
  pltpu.sync_copy(srci_hbm.at[tid], srcv)
  pltpu.sync_copy(dsti_hbm.at[tid], dstv)
  plsc.subcore_barrier()

  bufs = (b0, b1)
  sems = (sem0, sem1)
  for b in range(2):
    pltpu.async_copy(g_hbm.at[srcv.at[b]], bufs[b], sems[b])

  def body(grp, carry):
    for b in range(2):
      j = grp * 2 + b
      pltpu.make_async_copy(g_hbm.at[srcv.at[j]], bufs[b], sems[b]).wait()
      pltpu.sync_copy(bufs[b], agg_sh.at[dstv.at[j]], add=True)

      @pl.when(j + 2 < NCHUNK)
      def _():
        pltpu.async_copy(g_hbm.at[srcv.at[j + 2]], bufs[b], sems[b])
    return carry

  lax.fori_loop(0, NCHUNK // 2, body, 0)
  plsc.subcore_barrier()
  pltpu.sync_copy(agg_sh.at[pl.ds(base, ROWS_PER_TILE)],
                  out_hbm.at[c].at[pl.ds(base, ROWS_PER_TILE)])


def _mlp_body(x_ref, w1_ref, b1_ref, w2_ref, b2_ref, w3_ref, b3_ref, o_ref):
  h = jnp.dot(x_ref[...], w1_ref[...], preferred_element_type=_f32)
  h = jnp.maximum(h + b1_ref[...], 0.0)
  h = jnp.dot(h, w2_ref[...], preferred_element_type=_f32)
  h = jnp.maximum(h + b2_ref[...], 0.0)
  o_ref[...] = jnp.dot(h, w3_ref[...], preferred_element_type=_f32) + b3_ref[...]


def _mlp(x_pad, W1, b1, W2, b2, W3, b3):
  nb = NPAD // RB
  full = lambda shape: pl.BlockSpec(shape, lambda i: tuple(0 for _ in shape))
  return pl.pallas_call(
      _mlp_body,
      grid=(nb,),
      in_specs=[
          pl.BlockSpec((RB, IN_CH), lambda i: (i, 0)),
          full((IN_CH, HID_CH)), full((1, HID_CH)),
          full((HID_CH, HID_CH)), full((1, HID_CH)),
          full((HID_CH, OUT_CH)), full((1, OUT_CH)),
      ],
      out_specs=pl.BlockSpec((RB, OUT_CH), lambda i: (i, 0)),
      out_shape=jax.ShapeDtypeStruct((NPAD, OUT_CH), _f32),
  )(x_pad, W1, b1, W2, b2, W3, b3)


def _prep_body(h_ref, deg_ref, g0_ref, c0_ref, c1_ref, fd_ref, ah0_ref):
  h = h_ref[...]
  dsrc = deg_ref[0, 0, :, 0:1] + deg_ref[1, 0, :, 0:1]
  ddst = deg_ref[0, 1, :, 0:1] + deg_ref[1, 1, :, 0:1]
  rs = lax.rsqrt(jnp.maximum(dsrc, 1.0))
  rd = lax.rsqrt(jnp.maximum(ddst, 1.0))
  g0_ref[...] = rs * h
  ah0_ref[...] = ALPHA * h
  c0_ref[...] = (ALPHA * rs) * h
  c1_ref[...] = jnp.broadcast_to((1.0 - ALPHA) * rs * rd, h.shape)
  fd_ref[...] = jnp.broadcast_to((1.0 - ALPHA) * rd, h.shape)


def _prep(h_pad, degs):
  nb = NPAD // RB
  sds = jax.ShapeDtypeStruct((NPAD, OUT_CH), _f32)
  return pl.pallas_call(
      _prep_body,
      grid=(nb,),
      in_specs=[
          pl.BlockSpec((RB, OUT_CH), lambda i: (i, 0)),
          pl.BlockSpec((NC, 2, RB, OUT_CH), lambda i: (0, 0, i, 0)),
      ],
      out_specs=[pl.BlockSpec((RB, OUT_CH), lambda i: (i, 0))] * 5,
      out_shape=[sds] * 5,
  )(h_pad, degs)


def _combine_body(s_ref, c1_ref, c0_ref, g_ref):
  g_ref[...] = c1_ref[...] * (s_ref[0] + s_ref[1]) + c0_ref[...]


def _combine(S2, c1f, c0):
  nb = NPAD // RB
  return pl.pallas_call(
      _combine_body,
      grid=(nb,),
      in_specs=[
          pl.BlockSpec((NC, RB, OUT_CH), lambda i: (0, i, 0)),
          pl.BlockSpec((RB, OUT_CH), lambda i: (i, 0)),
          pl.BlockSpec((RB, OUT_CH), lambda i: (i, 0)),
      ],
      out_specs=pl.BlockSpec((RB, OUT_CH), lambda i: (i, 0)),
      out_shape=jax.ShapeDtypeStruct((NPAD, OUT_CH), _f32),
  )(S2, c1f, c0)


def _final_body(s_ref, fd_ref, ah0_ref, o_ref):
  z = fd_ref[...] * (s_ref[0] + s_ref[1]) + ah0_ref[...]
  m = jnp.max(z, axis=1, keepdims=True)
  lse = jnp.log(jnp.sum(jnp.exp(z - m), axis=1, keepdims=True)) + m
  o_ref[...] = z - lse


def _final(S2, fd, ah0):
  fb = 1000
  nb = N_NODES // fb
  return pl.pallas_call(
      _final_body,
      grid=(nb,),
      in_specs=[
          pl.BlockSpec((NC, fb, OUT_CH), lambda i: (0, i, 0)),
          pl.BlockSpec((fb, OUT_CH), lambda i: (i, 0)),
          pl.BlockSpec((fb, OUT_CH), lambda i: (i, 0)),
      ],
      out_specs=pl.BlockSpec((fb, OUT_CH), lambda i: (i, 0)),
      out_shape=jax.ShapeDtypeStruct((N_NODES, OUT_CH), _f32),
  )(S2, fd, ah0)


def kernel(x, edge_index, W1, b1, W2, b2, W3, b3):
  x_pad = jnp.concatenate(
      [x, jnp.zeros((NPAD - N_NODES, IN_CH), _f32)], axis=0)
  pad_n = NT * EPT - N_EDGES
  src = edge_index[0].astype(_i32)
  dst = edge_index[1].astype(_i32)
  pad_idx = jnp.full((pad_n,), N_NODES, _i32)
  srci = jnp.concatenate([src, pad_idx]).reshape(NT, NCHUNK, CH)
  dsti = jnp.concatenate([dst, pad_idx]).reshape(NT, NCHUNK, CH)
  zrs = jnp.zeros((NPAD, OUT_CH), _f32)

  h_pad = _mlp(x_pad, W1, b1.reshape(1, HID_CH), W2, b2.reshape(1, HID_CH),
               W3, b3.reshape(1, OUT_CH))
  degs = _deg_kernel(srci, dsti, zrs)
  g, c0, c1f, fd, ah0 = _prep(h_pad, degs)
  for _ in range(K_PROP - 1):
    S2 = _step_kernel(g, srci, dsti, zrs)
    g = _combine(S2, c1f, c0)
  S2 = _step_kernel(g, srci, dsti, zrs)
  return _final(S2, fd, ah0)
